# Initial kernel scaffold; baseline (speedup 1.0000x reference)
import jax
import jax.numpy as jnp
from jax.experimental import pallas as pl


def kernel(x, edge_index, batch, W_pre, b_pre, W1, att_src1, att_dst1, bias1, W2, att_src2, att_dst2, bias2, Wp1, bp1, Wp2, bp2):
    raise NotImplementedError("write your pallas kernel here")


# speedup vs baseline: 22.1869x; 22.1869x over previous
import functools

import jax
import jax.numpy as jnp
from jax import lax
from jax.experimental import pallas as pl
from jax.experimental.pallas import tpu as pltpu
from jax.experimental.pallas import tpu_sc as plsc

N = 10000
D = 128
H = 2
C = 64
G = 64
NC = 10
NB = 512
NP = 10240
NT = 10016
E0 = 320000
EE = E0 + N
CH = 2000
EP = 332000
EPS = 1e-16
NEG = -1e30


def _tc_layer(xt, wt, bias_t, am, pone, den=None, pre_w=None):
    grid = NP // NB
    have_den = den is not None

    def body(*refs):
        if have_den:
            xt_ref, den_ref, wt_ref, b_ref, am_ref, po_ref, xl_ref, at_ref, pool_ref, mx_ref = refs
        else:
            xt_ref, pw_ref, wt_ref, b_ref, am_ref, po_ref, xl_ref, at_ref, pool_ref, mx_ref = refs
        i = pl.program_id(0)
        if have_den:
            xv = xt_ref[...]
            d0 = den_ref[0:1, :] + EPS
            d1 = den_ref[1:2, :] + EPS
            xv = jnp.concatenate([xv[0:64, :] / d0, xv[64:128, :] / d1], axis=0)
        else:
            xv = jnp.dot(pw_ref[...], xt_ref[...],
                         preferred_element_type=jnp.float32)
        b = b_ref[:, 0:1]
        x = jnp.maximum(xv + b, 0.0)
        xl_ref[...] = jnp.dot(wt_ref[...], x, preferred_element_type=jnp.float32)
        at = jnp.dot(am_ref[...], x, preferred_element_type=jnp.float32)
        at_ref[...] = at

        @pl.when(i == 0)
        def _():
            pool_ref[...] = jnp.zeros_like(pool_ref)
            mx_ref[...] = jnp.full_like(mx_ref, NEG)

        pool_ref[...] += lax.dot_general(
            po_ref[...], x, (((0,), (1,)), ((), ())),
            preferred_element_type=jnp.float32)
        m = jnp.maximum(jnp.maximum(at[:, 0:128], at[:, 128:256]),
                        jnp.maximum(at[:, 256:384], at[:, 384:512]))
        mx_ref[...] = jnp.maximum(mx_ref[...], m)

    in_specs = [pl.BlockSpec((D, NB), lambda i: (0, i))]
    args = [xt]
    if have_den:
        in_specs.append(pl.BlockSpec((8, NB), lambda i: (0, i)))
        args.append(den)
    else:
        in_specs.append(pl.BlockSpec((D, D), lambda i: (0, 0)))
        args.append(pre_w)
    in_specs += [
        pl.BlockSpec((D, D), lambda i: (0, 0)),
        pl.BlockSpec((D, 128), lambda i: (0, 0)),
        pl.BlockSpec((8, D), lambda i: (0, 0)),
        pl.BlockSpec((NB, G), lambda i: (i, 0)),
    ]
    args += [wt, bias_t, am, pone]
    out_shape = [
        jax.ShapeDtypeStruct((D, NP), jnp.float32),
        jax.ShapeDtypeStruct((8, NP), jnp.float32),
        jax.ShapeDtypeStruct((G, D), jnp.float32),
        jax.ShapeDtypeStruct((8, 128), jnp.float32),
    ]
    out_specs = [
        pl.BlockSpec((D, NB), lambda i: (0, i)),
        pl.BlockSpec((8, NB), lambda i: (0, i)),
        pl.BlockSpec((G, D), lambda i: (0, 0)),
        pl.BlockSpec((8, 128), lambda i: (0, 0)),
    ]
    return pl.pallas_call(
        body, grid=(grid,), in_specs=in_specs, out_specs=out_specs,
        out_shape=out_shape)(*args)


def _tc_final_pool(xt, den, bias_t, pone):
    grid = NP // NB

    def body(xt_ref, den_ref, b_ref, po_ref, pool_ref):
        i = pl.program_id(0)
        xv = xt_ref[...]
        d0 = den_ref[0:1, :] + EPS
        d1 = den_ref[1:2, :] + EPS
        xv = jnp.concatenate([xv[0:64, :] / d0, xv[64:128, :] / d1], axis=0)
        x = jnp.maximum(xv + b_ref[:, 0:1], 0.0)

        @pl.when(i == 0)
        def _():
            pool_ref[...] = jnp.zeros_like(pool_ref)

        pool_ref[...] += lax.dot_general(
            po_ref[...], x, (((0,), (1,)), ((), ())),
            preferred_element_type=jnp.float32)

    return pl.pallas_call(
        body, grid=(grid,),
        in_specs=[
            pl.BlockSpec((D, NB), lambda i: (0, i)),
            pl.BlockSpec((8, NB), lambda i: (0, i)),
            pl.BlockSpec((D, 128), lambda i: (0, 0)),
            pl.BlockSpec((NB, G), lambda i: (i, 0)),
        ],
        out_specs=pl.BlockSpec((G, D), lambda i: (0, 0)),
        out_shape=jax.ShapeDtypeStruct((G, D), jnp.float32),
    )(xt, den, bias_t, pone)


def _tc_head(p0, p1, p2, wp1, bp1r, wp2p, bp2r):
    def body(p0_ref, p1_ref, p2_ref, w1_ref, b1_ref, w2_ref, b2_ref, out_ref):
        e = p0_ref[...] + p1_ref[...] + p2_ref[...]
        h = jnp.maximum(
            jnp.dot(e, w1_ref[...], preferred_element_type=jnp.float32)
            + b1_ref[0:1, :], 0.0)
        pred = (jnp.dot(h, w2_ref[...], preferred_element_type=jnp.float32)
                + b2_ref[0:1, :])
        mask = lax.broadcasted_iota(jnp.int32, (G, 128), 1) < NC
        pm = jnp.where(mask, pred, NEG)
        mx = jnp.max(pm, axis=1, keepdims=True)
        s = jnp.sum(jnp.where(mask, jnp.exp(pm - mx), 0.0), axis=1, keepdims=True)
        out_ref[...] = pm - mx - jnp.log(s)

    return pl.pallas_call(
        body,
        out_shape=jax.ShapeDtypeStruct((G, 128), jnp.float32),
    )(p0, p1, p2, wp1, bp1r, wp2p, bp2r)


def _sc_edge(xlt, attn, src_p, dst_p, mrow):
    mesh = plsc.VectorSubcoreMesh(core_axis_name="c", subcore_axis_name="s")
    xlt1 = xlt.reshape(D * NP)
    attn1 = attn.reshape(8 * NP)
    mrow1 = mrow.reshape(32)

    @functools.partial(
        pl.kernel,
        out_type=(jax.ShapeDtypeStruct((D * NP,), jnp.float32),
                  jax.ShapeDtypeStruct((8 * NP,), jnp.float32)),
        mesh=mesh,
        compiler_params=pltpu.CompilerParams(needs_layout_passes=False),
        scratch_types=[
            pltpu.VMEM((NT,), jnp.float32),
            pltpu.VMEM((NT,), jnp.float32),
            pltpu.VMEM((NT,), jnp.float32),
            pltpu.VMEM((NT,), jnp.float32),
            pltpu.VMEM((NP,), jnp.float32),
            pltpu.VMEM((NP,), jnp.float32),
            pltpu.VMEM((NP,), jnp.float32),
            pltpu.VMEM((NP,), jnp.float32),
            pltpu.VMEM((NT,), jnp.float32),
            pltpu.VMEM((NT,), jnp.float32),
            pltpu.VMEM((NP,), jnp.float32),
            pltpu.VMEM((CH,), jnp.int32),
            pltpu.VMEM((CH,), jnp.int32),
            pltpu.VMEM((16,), jnp.float32),
        ],
    )
    def k(xlt_h, attn_h, src_h, dst_h, mrow_h,
          acc_out, den_out,
          t0, t1, t2, t3, a0, a1_, a2_, a3, asrc_v, adst_v, den_v,
          srcb, dstb, mvec_v):
        tbl = (t0, t1, t2, t3)
        acc = (a0, a1_, a2_, a3)
        hh = lax.axis_index("c")
        tt = lax.axis_index("s")
        row0 = 64 * hh + 4 * tt

        for c in range(4):
            pltpu.sync_copy(xlt_h.at[pl.ds((row0 + c) * NP, NT)], tbl[c])
        pltpu.sync_copy(attn_h.at[pl.ds(hh * NP, NT)], asrc_v)
        pltpu.sync_copy(attn_h.at[pl.ds((2 + hh) * NP, NT)], adst_v)
        pltpu.sync_copy(mrow_h.at[pl.ds(hh * 16, 16)], mvec_v)

        def zbody(i, _):
            z = jnp.zeros((16,), jnp.float32)
            for c in range(4):
                acc[c][pl.ds(i * 16, 16)] = z
            den_v[pl.ds(i * 16, 16)] = z
            return 0

        lax.fori_loop(0, NP // 16, zbody, 0)
        mv = mvec_v[...]

        def chunk_body(g, _):
            pltpu.sync_copy(src_h.at[pl.ds(g * CH, CH)], srcb)
            pltpu.sync_copy(dst_h.at[pl.ds(g * CH, CH)], dstb)

            def ebody(j, _2):
                sv = srcb[pl.ds(j * 16, 16)]
                dv = dstb[pl.ds(j * 16, 16)]
                a1 = plsc.load_gather(asrc_v, [sv])
                a2 = plsc.load_gather(adst_v, [dv])
                raw = a1 + a2
                lr2 = jnp.where(raw >= 0, raw, raw * 0.2)
                w = jnp.exp(lr2 - mv)
                plsc.addupdate_scatter(den_v, [dv], w)
                for c in range(4):
                    gch = plsc.load_gather(tbl[c], [sv])
                    plsc.addupdate_scatter(acc[c], [dv], gch * w)
                return 0

            lax.fori_loop(0, CH // 16, ebody, 0)
            return 0

        lax.fori_loop(0, EP // CH, chunk_body, 0)

        z = jnp.zeros((16,), jnp.float32)
        for c in range(4):
            acc[c][pl.ds(N, 16)] = z
        den_v[pl.ds(N, 16)] = z

        for c in range(4):
            pltpu.sync_copy(acc[c], acc_out.at[pl.ds((row0 + c) * NP, NP)])

        @pl.when(tt == 0)
        def _():
            pltpu.sync_copy(den_v, den_out.at[pl.ds(hh * NP, NP)])

    acc1, den1 = k(xlt1, attn1, src_p, dst_p, mrow1)
    return acc1.reshape(D, NP), den1.reshape(8, NP)


def _attn_mat(w, att_src, att_dst):
    wr = w.reshape(D, H, C)
    vs = jnp.einsum("dhc,hc->hd", wr, att_src)
    vd = jnp.einsum("dhc,hc->hd", wr, att_dst)
    return jnp.concatenate([vs, vd, jnp.zeros((4, D), jnp.float32)], axis=0)


def _mshift(mx):
    mxv = jnp.max(mx, axis=1)
    ub = mxv[:2] + mxv[2:4]
    m = jnp.where(ub >= 0, ub, 0.2 * ub)
    return jnp.broadcast_to(m[:, None], (2, 16)).astype(jnp.float32)


def kernel(x, edge_index, batch, W_pre, b_pre, W1, att_src1, att_dst1, bias1,
           W2, att_src2, att_dst2, bias2, Wp1, bp1, Wp2, bp2):
    xt = jnp.zeros((D, NP), jnp.float32).at[:, :N].set(x.T)
    loop = jnp.arange(N, dtype=jnp.int32)
    src = jnp.concatenate([edge_index[0], loop,
                           jnp.full((EP - EE,), N, jnp.int32)])
    dst = jnp.concatenate([edge_index[1], loop,
                           jnp.full((EP - EE,), N, jnp.int32)])
    pone = jnp.zeros((NP, G), jnp.float32).at[:N].set(
        (batch[:, None] == jnp.arange(G)[None, :]).astype(jnp.float32))
    bcol_pre = jnp.broadcast_to(b_pre[:, None], (D, 128))
    bcol1 = jnp.broadcast_to(bias1[:, None], (D, 128))
    bcol2 = jnp.broadcast_to(bias2[:, None], (D, 128))
    am1 = _attn_mat(W1, att_src1, att_dst1)
    am2 = _attn_mat(W2, att_src2, att_dst2)
    wp2p = jnp.zeros((D, 128), jnp.float32).at[:, :NC].set(Wp2)
    bp1r = jnp.broadcast_to(bp1[None, :], (8, D))
    bp2r = jnp.zeros((8, 128), jnp.float32).at[:, :NC].set(
        jnp.broadcast_to(bp2[None, :], (8, NC)))

    xl1, attn1, pool0, mx1 = _tc_layer(xt, W1.T, bcol_pre, am1, pone,
                                       pre_w=W_pre.T)
    mrow1 = _mshift(mx1)

    acc1, den1 = _sc_edge(xl1, attn1, src, dst, mrow1)

    xl2, attn2, pool1, mx2 = _tc_layer(acc1, W2.T, bcol1, am2, pone, den=den1)
    mrow2 = _mshift(mx2)

    acc2, den2 = _sc_edge(xl2, attn2, src, dst, mrow2)

    pool2 = _tc_final_pool(acc2, den2, bcol2, pone)

    out = _tc_head(pool0, pool1, pool2, Wp1, bp1r, wp2p, bp2r)
    return out[:, :NC]


# speedup vs baseline: 26.9744x; 1.2158x over previous
import functools

import jax
import jax.numpy as jnp
from jax import lax
from jax.experimental import pallas as pl
from jax.experimental.pallas import tpu as pltpu
from jax.experimental.pallas import tpu_sc as plsc

N = 10000
D = 128
H = 2
C = 64
G = 64
NC = 10
NB = 512
NP = 10240
NT = 10016
E0 = 320000
EE = E0 + N
CH = 3200
NCH = 104
EP = CH * NCH
EPS = 1e-16
NEG = -1e30


def _tc_layer(xt, wt, bias_t, am, pone, den=None, pre_w=None):
    grid = NP // NB
    have_den = den is not None

    def body(*refs):
        if have_den:
            xt_ref, den_ref, wt_ref, b_ref, am_ref, po_ref, xl_ref, at_ref, pool_ref, mx_ref = refs
        else:
            xt_ref, pw_ref, wt_ref, b_ref, am_ref, po_ref, xl_ref, at_ref, pool_ref, mx_ref = refs
        i = pl.program_id(0)
        if have_den:
            xv = xt_ref[...]
            d0 = den_ref[0:1, :] + EPS
            d1 = den_ref[1:2, :] + EPS
            xv = jnp.concatenate([xv[0:64, :] / d0, xv[64:128, :] / d1], axis=0)
        else:
            xv = jnp.dot(pw_ref[...], xt_ref[...],
                         preferred_element_type=jnp.float32)
        b = b_ref[:, 0:1]
        x = jnp.maximum(xv + b, 0.0)
        xl_ref[...] = jnp.dot(wt_ref[...], x, preferred_element_type=jnp.float32)
        at = jnp.dot(am_ref[...], x, preferred_element_type=jnp.float32)
        at_ref[...] = at

        @pl.when(i == 0)
        def _():
            pool_ref[...] = jnp.zeros_like(pool_ref)
            mx_ref[...] = jnp.full_like(mx_ref, NEG)

        pool_ref[...] += lax.dot_general(
            po_ref[...], x, (((0,), (1,)), ((), ())),
            preferred_element_type=jnp.float32)
        m = jnp.maximum(jnp.maximum(at[:, 0:128], at[:, 128:256]),
                        jnp.maximum(at[:, 256:384], at[:, 384:512]))
        mx_ref[...] = jnp.maximum(mx_ref[...], m)

    in_specs = [pl.BlockSpec((D, NB), lambda i: (0, i))]
    args = [xt]
    if have_den:
        in_specs.append(pl.BlockSpec((8, NB), lambda i: (0, i)))
        args.append(den)
    else:
        in_specs.append(pl.BlockSpec((D, D), lambda i: (0, 0)))
        args.append(pre_w)
    in_specs += [
        pl.BlockSpec((D, D), lambda i: (0, 0)),
        pl.BlockSpec((D, 128), lambda i: (0, 0)),
        pl.BlockSpec((8, D), lambda i: (0, 0)),
        pl.BlockSpec((NB, G), lambda i: (i, 0)),
    ]
    args += [wt, bias_t, am, pone]
    out_shape = [
        jax.ShapeDtypeStruct((D, NP), jnp.float32),
        jax.ShapeDtypeStruct((8, NP), jnp.float32),
        jax.ShapeDtypeStruct((G, D), jnp.float32),
        jax.ShapeDtypeStruct((8, 128), jnp.float32),
    ]
    out_specs = [
        pl.BlockSpec((D, NB), lambda i: (0, i)),
        pl.BlockSpec((8, NB), lambda i: (0, i)),
        pl.BlockSpec((G, D), lambda i: (0, 0)),
        pl.BlockSpec((8, 128), lambda i: (0, 0)),
    ]
    return pl.pallas_call(
        body, grid=(grid,), in_specs=in_specs, out_specs=out_specs,
        out_shape=out_shape)(*args)


def _tc_final_pool(xt, den, bias_t, pone):
    grid = NP // NB

    def body(xt_ref, den_ref, b_ref, po_ref, pool_ref):
        i = pl.program_id(0)
        xv = xt_ref[...]
        d0 = den_ref[0:1, :] + EPS
        d1 = den_ref[1:2, :] + EPS
        xv = jnp.concatenate([xv[0:64, :] / d0, xv[64:128, :] / d1], axis=0)
        x = jnp.maximum(xv + b_ref[:, 0:1], 0.0)

        @pl.when(i == 0)
        def _():
            pool_ref[...] = jnp.zeros_like(pool_ref)

        pool_ref[...] += lax.dot_general(
            po_ref[...], x, (((0,), (1,)), ((), ())),
            preferred_element_type=jnp.float32)

    return pl.pallas_call(
        body, grid=(grid,),
        in_specs=[
            pl.BlockSpec((D, NB), lambda i: (0, i)),
            pl.BlockSpec((8, NB), lambda i: (0, i)),
            pl.BlockSpec((D, 128), lambda i: (0, 0)),
            pl.BlockSpec((NB, G), lambda i: (i, 0)),
        ],
        out_specs=pl.BlockSpec((G, D), lambda i: (0, 0)),
        out_shape=jax.ShapeDtypeStruct((G, D), jnp.float32),
    )(xt, den, bias_t, pone)


def _tc_head(p0, p1, p2, wp1, bp1r, wp2p, bp2r):
    def body(p0_ref, p1_ref, p2_ref, w1_ref, b1_ref, w2_ref, b2_ref, out_ref):
        e = p0_ref[...] + p1_ref[...] + p2_ref[...]
        h = jnp.maximum(
            jnp.dot(e, w1_ref[...], preferred_element_type=jnp.float32)
            + b1_ref[0:1, :], 0.0)
        pred = (jnp.dot(h, w2_ref[...], preferred_element_type=jnp.float32)
                + b2_ref[0:1, :])
        mask = lax.broadcasted_iota(jnp.int32, (G, 128), 1) < NC
        pm = jnp.where(mask, pred, NEG)
        mx = jnp.max(pm, axis=1, keepdims=True)
        s = jnp.sum(jnp.where(mask, jnp.exp(pm - mx), 0.0), axis=1, keepdims=True)
        out_ref[...] = pm - mx - jnp.log(s)

    return pl.pallas_call(
        body,
        out_shape=jax.ShapeDtypeStruct((G, 128), jnp.float32),
    )(p0, p1, p2, wp1, bp1r, wp2p, bp2r)


def _sc_edge(xlt, attn, src_p, dst_p, mrow):
    mesh = plsc.VectorSubcoreMesh(core_axis_name="c", subcore_axis_name="s")
    xlt1 = xlt.reshape(D * NP)
    attn1 = attn.reshape(8 * NP)
    mrow1 = mrow.reshape(32)

    @functools.partial(
        pl.kernel,
        out_type=(jax.ShapeDtypeStruct((D * NP,), jnp.float32),
                  jax.ShapeDtypeStruct((8 * NP,), jnp.float32)),
        mesh=mesh,
        compiler_params=pltpu.CompilerParams(needs_layout_passes=False),
        scratch_types=[
            pltpu.VMEM((NT,), jnp.float32),
            pltpu.VMEM((NT,), jnp.float32),
            pltpu.VMEM((NT,), jnp.float32),
            pltpu.VMEM((NT,), jnp.float32),
            pltpu.VMEM((NP,), jnp.float32),
            pltpu.VMEM((NP,), jnp.float32),
            pltpu.VMEM((NP,), jnp.float32),
            pltpu.VMEM((NP,), jnp.float32),
            pltpu.VMEM((NT,), jnp.float32),
            pltpu.VMEM((NT,), jnp.float32),
            pltpu.VMEM((NP,), jnp.float32),
            pltpu.VMEM((CH,), jnp.int32),
            pltpu.VMEM((CH,), jnp.int32),
            pltpu.VMEM((CH,), jnp.int32),
            pltpu.VMEM((CH,), jnp.int32),
            pltpu.VMEM((16,), jnp.float32),
            pltpu.SemaphoreType.DMA,
            pltpu.SemaphoreType.DMA,
            pltpu.SemaphoreType.DMA,
            pltpu.SemaphoreType.DMA,
        ],
    )
    def k(xlt_h, attn_h, src_h, dst_h, mrow_h,
          acc_out, den_out,
          t0, t1, t2, t3, a0, a1_, a2_, a3, asrc_v, adst_v, den_v,
          s0b, s1b, d0b, d1b, mvec_v, ss0, ss1, ds0, ds1):
        tbl = (t0, t1, t2, t3)
        acc = (a0, a1_, a2_, a3)
        hh = lax.axis_index("c")
        tt = lax.axis_index("s")
        row0 = 64 * hh + 4 * tt

        for c in range(4):
            pltpu.sync_copy(xlt_h.at[pl.ds((row0 + c) * NP, NT)], tbl[c])
        pltpu.sync_copy(attn_h.at[pl.ds(hh * NP, NT)], asrc_v)
        pltpu.sync_copy(attn_h.at[pl.ds((2 + hh) * NP, NT)], adst_v)
        pltpu.sync_copy(mrow_h.at[pl.ds(hh * 16, 16)], mvec_v)

        def zbody(i, _):
            z = jnp.zeros((16,), jnp.float32)
            for c in range(4):
                acc[c][pl.ds(i * 16, 16)] = z
            den_v[pl.ds(i * 16, 16)] = z
            return 0

        lax.fori_loop(0, NP // 16, zbody, 0)
        mv = mvec_v[...]

        bufs = ((s0b, d0b, ss0, ds0), (s1b, d1b, ss1, ds1))
        for b in range(2):
            pltpu.async_copy(src_h.at[pl.ds(b * CH, CH)], bufs[b][0], bufs[b][2])
            pltpu.async_copy(dst_h.at[pl.ds(b * CH, CH)], bufs[b][1], bufs[b][3])

        def chunk_body(i, _):
            for b in range(2):
                g = 2 * i + b
                sb, db, ssem, dsem = bufs[b]
                pltpu.make_async_copy(src_h.at[pl.ds(g * CH, CH)], sb, ssem).wait()
                pltpu.make_async_copy(dst_h.at[pl.ds(g * CH, CH)], db, dsem).wait()

                def ebody(j, _2):
                    sv = sb[pl.ds(j * 16, 16)]
                    dv = db[pl.ds(j * 16, 16)]
                    a1 = plsc.load_gather(asrc_v, [sv])
                    a2 = plsc.load_gather(adst_v, [dv])
                    raw = a1 + a2
                    lr2 = jnp.where(raw >= 0, raw, raw * 0.2)
                    w = jnp.exp(lr2 - mv)
                    plsc.addupdate_scatter(den_v, [dv], w)
                    for c in range(4):
                        gch = plsc.load_gather(tbl[c], [sv])
                        plsc.addupdate_scatter(acc[c], [dv], gch * w)
                    return 0

                lax.fori_loop(0, CH // 16, ebody, 0, unroll=4)

                @pl.when(g + 2 < NCH)
                def _():
                    pltpu.async_copy(
                        src_h.at[pl.ds((g + 2) * CH, CH)], sb, ssem)
                    pltpu.async_copy(
                        dst_h.at[pl.ds((g + 2) * CH, CH)], db, dsem)
            return 0

        lax.fori_loop(0, NCH // 2, chunk_body, 0)

        z = jnp.zeros((16,), jnp.float32)
        for c in range(4):
            acc[c][pl.ds(N, 16)] = z
        den_v[pl.ds(N, 16)] = z

        for c in range(4):
            pltpu.sync_copy(acc[c], acc_out.at[pl.ds((row0 + c) * NP, NP)])

        @pl.when(tt == 0)
        def _():
            pltpu.sync_copy(den_v, den_out.at[pl.ds(hh * NP, NP)])

    acc1, den1 = k(xlt1, attn1, src_p, dst_p, mrow1)
    return acc1.reshape(D, NP), den1.reshape(8, NP)


def _attn_mat(w, att_src, att_dst):
    wr = w.reshape(D, H, C)
    vs = jnp.einsum("dhc,hc->hd", wr, att_src)
    vd = jnp.einsum("dhc,hc->hd", wr, att_dst)
    return jnp.concatenate([vs, vd, jnp.zeros((4, D), jnp.float32)], axis=0)


def _mshift(mx):
    mxv = jnp.max(mx, axis=1)
    ub = mxv[:2] + mxv[2:4]
    m = jnp.where(ub >= 0, ub, 0.2 * ub)
    return jnp.broadcast_to(m[:, None], (2, 16)).astype(jnp.float32)


def kernel(x, edge_index, batch, W_pre, b_pre, W1, att_src1, att_dst1, bias1,
           W2, att_src2, att_dst2, bias2, Wp1, bp1, Wp2, bp2):
    xt = jnp.zeros((D, NP), jnp.float32).at[:, :N].set(x.T)
    loop = jnp.arange(N, dtype=jnp.int32)
    src = jnp.concatenate([edge_index[0], loop,
                           jnp.full((EP - EE,), N, jnp.int32)])
    dst = jnp.concatenate([edge_index[1], loop,
                           jnp.full((EP - EE,), N, jnp.int32)])
    pone = jnp.zeros((NP, G), jnp.float32).at[:N].set(
        (batch[:, None] == jnp.arange(G)[None, :]).astype(jnp.float32))
    bcol_pre = jnp.broadcast_to(b_pre[:, None], (D, 128))
    bcol1 = jnp.broadcast_to(bias1[:, None], (D, 128))
    bcol2 = jnp.broadcast_to(bias2[:, None], (D, 128))
    am1 = _attn_mat(W1, att_src1, att_dst1)
    am2 = _attn_mat(W2, att_src2, att_dst2)
    wp2p = jnp.zeros((D, 128), jnp.float32).at[:, :NC].set(Wp2)
    bp1r = jnp.broadcast_to(bp1[None, :], (8, D))
    bp2r = jnp.zeros((8, 128), jnp.float32).at[:, :NC].set(
        jnp.broadcast_to(bp2[None, :], (8, NC)))

    xl1, attn1, pool0, mx1 = _tc_layer(xt, W1.T, bcol_pre, am1, pone,
                                       pre_w=W_pre.T)
    mrow1 = _mshift(mx1)

    acc1, den1 = _sc_edge(xl1, attn1, src, dst, mrow1)

    xl2, attn2, pool1, mx2 = _tc_layer(acc1, W2.T, bcol1, am2, pone, den=den1)
    mrow2 = _mshift(mx2)

    acc2, den2 = _sc_edge(xl2, attn2, src, dst, mrow2)

    pool2 = _tc_final_pool(acc2, den2, bcol2, pone)

    out = _tc_head(pool0, pool1, pool2, Wp1, bp1r, wp2p, bp2r)
    return out[:, :NC]
